# baseline (device time: 28415 ns/iter reference)
import functools

import jax
import jax.numpy as jnp
from jax import lax
from jax.experimental import pallas as pl
from jax.experimental.pallas import tpu as pltpu

N_DEV = 4
SQ = 256
D_MODEL = 1024
CPC = D_MODEL // N_DEV
DH = 128
H_LOC = 8
SCALE = 0.08838834764831843
F32 = jnp.float32
BF16 = jnp.bfloat16


def kernel(x, Wq, Wo, Wk, Wv):
    def body(x_ref, wq_ref, wo_ref, wk_ref, wv_ref, out_ref,
             wv_v, wo_v, send_ref, rs_ref, own_ref, ownb_ref, ag_ref,
             fetch_sems, rs_send, rs_recv, ag_send, ag_recv):
        d = lax.axis_index("i")

        cp_v = pltpu.make_async_copy(wv_ref, wv_v, fetch_sems.at[0])
        cp_v.start()
        cp_o = pltpu.make_async_copy(wo_ref, wo_v, fetch_sems.at[1])
        cp_o.start()

        barrier_sem = pltpu.get_barrier_semaphore()
        for o in (1, 2, 3):
            pl.semaphore_signal(
                barrier_sem, inc=1,
                device_id=((d + o) % N_DEV,),
                device_id_type=pl.DeviceIdType.MESH,
            )
        pl.semaphore_wait(barrier_sem, 3)

        xm = x_ref[0]
        q = jnp.dot(xm, wq_ref[...], preferred_element_type=F32)
        k = jnp.dot(xm, wk_ref[...], preferred_element_type=F32)
        cp_v.wait()
        v = jnp.dot(xm, wv_v[...], preferred_element_type=F32)

        o_heads = []
        for h in range(H_LOC):
            qh = q[:, h * DH:(h + 1) * DH]
            kh = k[:, h * DH:(h + 1) * DH]
            vh = v[:, h * DH:(h + 1) * DH]
            s = lax.dot_general(
                qh, kh, (((1,), (1,)), ((), ())),
                preferred_element_type=F32,
            ) * SCALE
            p = jnp.exp(s)
            l = jnp.sum(p, axis=1, keepdims=True)
            o_heads.append(jnp.dot(p, vh, preferred_element_type=F32) / l)
        attn = jnp.concatenate(o_heads, axis=1)

        cp_o.wait()
        for c in range(N_DEV):
            chunk = jnp.dot(attn, wo_v[:, c * CPC:(c + 1) * CPC],
                            preferred_element_type=F32)

            @pl.when(c != d)
            def _(c=c, chunk=chunk):
                send_ref[c] = chunk.astype(BF16)
                slot = (c - d) % N_DEV - 1
                pltpu.make_async_remote_copy(
                    src_ref=send_ref.at[c],
                    dst_ref=rs_ref.at[slot],
                    send_sem=rs_send.at[slot],
                    recv_sem=rs_recv.at[slot],
                    device_id=(c,),
                    device_id_type=pl.DeviceIdType.MESH,
                ).start()

            @pl.when(c == d)
            def _(chunk=chunk):
                own_ref[...] = chunk

        def waiter(dst, sem):
            return pltpu.make_async_remote_copy(
                src_ref=send_ref.at[0], dst_ref=dst,
                send_sem=sem, recv_sem=sem,
                device_id=(d,), device_id_type=pl.DeviceIdType.MESH,
            )

        for j in range(N_DEV - 1):
            waiter(rs_ref.at[j], rs_recv.at[j]).wait_recv()
        own = (own_ref[...]
               + rs_ref[0].astype(F32)
               + rs_ref[1].astype(F32)
               + rs_ref[2].astype(F32))
        ownb_ref[...] = own.astype(BF16)

        ag_rdmas = []
        for o in (1, 2, 3):
            r = pltpu.make_async_remote_copy(
                src_ref=ownb_ref,
                dst_ref=ag_ref.at[o - 1],
                send_sem=ag_send.at[o - 1],
                recv_sem=ag_recv.at[o - 1],
                device_id=((d + o) % N_DEV,),
                device_id_type=pl.DeviceIdType.MESH,
            )
            r.start()
            ag_rdmas.append(r)

        for c in range(N_DEV):
            @pl.when(c == d)
            def _(c=c):
                out_ref[0, :, c * CPC:(c + 1) * CPC] = own

        for r in ag_rdmas:
            r.wait_recv()
        for c in range(N_DEV):
            @pl.when(c != d)
            def _(c=c):
                j = (d - c) % N_DEV - 1
                out_ref[0, :, c * CPC:(c + 1) * CPC] = ag_ref[j].astype(F32)

        for j in range(N_DEV - 1):
            waiter(rs_ref.at[j], rs_send.at[j]).wait_send()
        for r in ag_rdmas:
            r.wait_send()

    return pl.pallas_call(
        body,
        out_shape=jax.ShapeDtypeStruct((1, SQ, D_MODEL), F32),
        in_specs=[
            pl.BlockSpec(memory_space=pltpu.VMEM),
            pl.BlockSpec(memory_space=pltpu.VMEM),
            pl.BlockSpec(memory_space=pltpu.MemorySpace.HBM),
            pl.BlockSpec(memory_space=pltpu.VMEM),
            pl.BlockSpec(memory_space=pltpu.MemorySpace.HBM),
        ],
        out_specs=pl.BlockSpec(memory_space=pltpu.VMEM),
        scratch_shapes=[
            pltpu.VMEM((D_MODEL, D_MODEL), F32),
            pltpu.VMEM((D_MODEL, D_MODEL), F32),
            pltpu.VMEM((N_DEV, SQ, CPC), BF16),
            pltpu.VMEM((N_DEV - 1, SQ, CPC), BF16),
            pltpu.VMEM((SQ, CPC), F32),
            pltpu.VMEM((SQ, CPC), BF16),
            pltpu.VMEM((N_DEV - 1, SQ, CPC), BF16),
            pltpu.SemaphoreType.DMA((2,)),
            pltpu.SemaphoreType.DMA((N_DEV - 1,)),
            pltpu.SemaphoreType.DMA((N_DEV - 1,)),
            pltpu.SemaphoreType.DMA((N_DEV - 1,)),
            pltpu.SemaphoreType.DMA((N_DEV - 1,)),
        ],
        compiler_params=pltpu.CompilerParams(collective_id=0),
    )(x, Wq, Wo, Wk, Wv)


# device time: 26688 ns/iter; 1.0647x vs baseline; 1.0647x over previous
import jax
import jax.numpy as jnp
from jax import lax
from jax.experimental import pallas as pl
from jax.experimental.pallas import tpu as pltpu

N_DEV = 4
SQ = 256
D_MODEL = 1024
CPC = D_MODEL // N_DEV
NW = 2
WC = CPC // NW
DH = 128
H_LOC = 8
SCALE = 0.08838834764831843
F32 = jnp.float32
BF16 = jnp.bfloat16


def kernel(x, Wq, Wo, Wk, Wv):
    def body(x_ref, wq_ref, wo_ref, wk_ref, wv_ref, out_ref,
             send_ref, rs_ref, own_ref, ownb_ref, ag_ref,
             rs_send, rs_recv, ag_send, ag_recv):
        d = lax.axis_index("i")

        barrier_sem = pltpu.get_barrier_semaphore()
        for o in (1, 2, 3):
            pl.semaphore_signal(
                barrier_sem, inc=1,
                device_id=((d + o) % N_DEV,),
                device_id_type=pl.DeviceIdType.MESH,
            )
        pl.semaphore_wait(barrier_sem, 3)

        xm = x_ref[0]
        q = jnp.dot(xm, wq_ref[...], preferred_element_type=F32)
        k = jnp.dot(xm, wk_ref[...], preferred_element_type=F32)
        v = jnp.dot(xm, wv_ref[...], preferred_element_type=F32)

        o_heads = []
        for h in range(H_LOC):
            qh = q[:, h * DH:(h + 1) * DH]
            kh = k[:, h * DH:(h + 1) * DH]
            vh = v[:, h * DH:(h + 1) * DH]
            s = lax.dot_general(
                qh, kh, (((1,), (1,)), ((), ())),
                preferred_element_type=F32,
            ) * SCALE
            p = jnp.exp(s)
            l = jnp.sum(p, axis=1, keepdims=True)
            o_heads.append(jnp.dot(p, vh, preferred_element_type=F32) / l)
        attn = jnp.concatenate(o_heads, axis=1)

        for c in range(N_DEV):
            chunk = jnp.dot(attn, wo_ref[:, c * CPC:(c + 1) * CPC],
                            preferred_element_type=F32)

            @pl.when(c != d)
            def _(c=c, chunk=chunk):
                slot = (c - d) % N_DEV - 1
                for w in range(NW):
                    send_ref[w, c] = chunk[:, w * WC:(w + 1) * WC].astype(BF16)
                    pltpu.make_async_remote_copy(
                        src_ref=send_ref.at[w, c],
                        dst_ref=rs_ref.at[w, slot],
                        send_sem=rs_send.at[w, slot],
                        recv_sem=rs_recv.at[w, slot],
                        device_id=(c,),
                        device_id_type=pl.DeviceIdType.MESH,
                    ).start()

            @pl.when(c == d)
            def _(chunk=chunk):
                own_ref[...] = chunk

        def waiter(dst, sem):
            return pltpu.make_async_remote_copy(
                src_ref=send_ref.at[0, 0], dst_ref=dst,
                send_sem=sem, recv_sem=sem,
                device_id=(d,), device_id_type=pl.DeviceIdType.MESH,
            )

        ag_rdmas = []
        for w in range(NW):
            for j in range(N_DEV - 1):
                waiter(rs_ref.at[w, j], rs_recv.at[w, j]).wait_recv()
            own_w = (own_ref[:, w * WC:(w + 1) * WC]
                     + rs_ref[w, 0].astype(F32)
                     + rs_ref[w, 1].astype(F32)
                     + rs_ref[w, 2].astype(F32))
            for c in range(N_DEV):
                @pl.when(c == d)
                def _(c=c, own_w=own_w, w=w):
                    out_ref[0, :, c * CPC + w * WC:c * CPC + (w + 1) * WC] = own_w
            ownb_ref[w] = own_w.astype(BF16)
            for o in (1, 2, 3):
                r = pltpu.make_async_remote_copy(
                    src_ref=ownb_ref.at[w],
                    dst_ref=ag_ref.at[w, o - 1],
                    send_sem=ag_send.at[w, o - 1],
                    recv_sem=ag_recv.at[w, o - 1],
                    device_id=((d + o) % N_DEV,),
                    device_id_type=pl.DeviceIdType.MESH,
                )
                r.start()
                ag_rdmas.append(r)

        for w in range(NW):
            for j in range(N_DEV - 1):
                waiter(ag_ref.at[w, j], ag_recv.at[w, j]).wait_recv()
            for c in range(N_DEV):
                @pl.when(c != d)
                def _(c=c, w=w):
                    j = (d - c) % N_DEV - 1
                    out_ref[0, :, c * CPC + w * WC:c * CPC + (w + 1) * WC] = (
                        ag_ref[w, j].astype(F32))

        for w in range(NW):
            for j in range(N_DEV - 1):
                waiter(rs_ref.at[w, j], rs_send.at[w, j]).wait_send()
        for r in ag_rdmas:
            r.wait_send()

    return pl.pallas_call(
        body,
        out_shape=jax.ShapeDtypeStruct((1, SQ, D_MODEL), F32),
        in_specs=[pl.BlockSpec(memory_space=pltpu.VMEM)] * 5,
        out_specs=pl.BlockSpec(memory_space=pltpu.VMEM),
        scratch_shapes=[
            pltpu.VMEM((NW, N_DEV, SQ, WC), BF16),
            pltpu.VMEM((NW, N_DEV - 1, SQ, WC), BF16),
            pltpu.VMEM((SQ, CPC), F32),
            pltpu.VMEM((NW, SQ, WC), BF16),
            pltpu.VMEM((NW, N_DEV - 1, SQ, WC), BF16),
            pltpu.SemaphoreType.DMA((NW, N_DEV - 1)),
            pltpu.SemaphoreType.DMA((NW, N_DEV - 1)),
            pltpu.SemaphoreType.DMA((NW, N_DEV - 1)),
            pltpu.SemaphoreType.DMA((NW, N_DEV - 1)),
        ],
        compiler_params=pltpu.CompilerParams(collective_id=0),
    )(x, Wq, Wo, Wk, Wv)
